# baseline (speedup 1.0000x reference)
import jax
import jax.numpy as jnp
from jax import lax
from jax.experimental import pallas as pl
from jax.experimental.pallas import tpu as pltpu
from jax.experimental.pallas import tpu_sc as plsc

D = 64
B = 1024
S = 200
NT = 20
SG = S - NT
V = 1000000
NC = 2
NS = 16
NW = NC * NS
BPW = B // NW
CH = 96
IW = 2 * CH
RV = NT + IW
NBUF = 4
GDEPTH = 2

VPAD = 1000064
NBLK = VPAD // 128
BPW0 = (NBLK + NW - 1) // NW


def _tp_body(wte_t_hbm, t128_hbm, tin, tout, isem, osem):
    w = lax.axis_index("s") * NC + lax.axis_index("c")
    base = w * BPW0
    lanes = lax.iota(jnp.int32, 16)

    def drain_o():
        pltpu.make_async_copy(
            t128_hbm.at[pl.ds(0, 128)], tout.at[pl.ds(0, 128)], osem).wait()

    def drain_i():
        pltpu.make_async_copy(
            t128_hbm.at[pl.ds(0, 64)], tin.at[pl.ds(0, 64)], isem).wait()

    @pl.when(base < NBLK)
    def _():
        pltpu.async_copy(wte_t_hbm.at[:, pl.ds(base * 128, 128)],
                         tin.at[pl.ds(0, 64)], isem)

    def body(i, carry):
        vh = base + i
        pin = lax.rem(i, 2)

        @pl.when(vh < NBLK)
        def _():
            drain_i()

            @pl.when((i < BPW0 - 1) & (vh + 1 < NBLK))
            def _():
                pltpu.async_copy(
                    wte_t_hbm.at[:, pl.ds((vh + 1) * 128, 128)],
                    tin.at[pl.ds(lax.rem(i + 1, 2) * 64, 64)], isem)

            @pl.when(i >= 2)
            def _():
                drain_o()

            pin64 = pin * 64
            pin128 = pin * 128

            @plsc.parallel_loop(0, 16, unroll=2)
            def ebody(j):
                dv0 = (lanes + j) & 15
                for r0 in (0, 16, 32, 48):
                    dv = dv0 + r0
                    dv2 = dv + pin64
                    vals = [plsc.load_gather(tin, [dv2, lanes + cb * 16])
                            for cb in range(8)]
                    for cb in range(8):
                        plsc.store_scatter(
                            tout, [lanes + cb * 16 + pin128, dv], vals[cb])
            pltpu.async_copy(tout.at[pl.ds(pin128, 128)],
                             t128_hbm.at[pl.ds(vh * 128, 128)], osem)

        return carry

    lax.fori_loop(0, BPW0, body, 0)
    for r in (2, 1):
        @pl.when(NBLK - base >= r)
        def _():
            drain_o()


def _sc_body(tok_hbm, t128_hbm, learned_hbm, out_hbm, idx_raw, idx_t, rows_v,
             gsem, osem):
    w = lax.axis_index("s") * NC + lax.axis_index("c")

    def drain_out():
        pltpu.make_async_copy(
            out_hbm.at[0], rows_v.at[0, pl.ds(0, S), pl.ds(0, D)],
            osem).wait()

    def drain_gather():
        pltpu.make_async_copy(
            t128_hbm.at[pl.ds(0, CH)], rows_v.at[0, pl.ds(NT, CH)],
            gsem).wait()

    pltpu.sync_copy(tok_hbm.at[pl.ds(NT, SG), pl.ds(w * BPW, BPW)], idx_raw)
    for k in range(NBUF):
        pltpu.sync_copy(learned_hbm, rows_v.at[k, pl.ds(0, NT), pl.ds(0, D)])

    lanes = lax.iota(jnp.int32, 16)

    def tbody(j, carry):
        colj = jnp.full((16,), j, jnp.int32)
        for k in range(IW // 16):
            rows = jnp.minimum(k * 16 + lanes, SG - 1)
            idx_t[j, pl.ds(k * 16, 16)] = plsc.load_gather(idx_raw, [rows, colj])
        return carry

    lax.fori_loop(0, BPW, tbody, 0)

    def body(j, carry):
        p = lax.rem(j, NBUF)

        @pl.when(j >= NBUF)
        def _():
            drain_out()

        @pl.when(j < BPW)
        def _():
            pltpu.async_copy(
                t128_hbm.at[idx_t.at[j, pl.ds(0, CH)]],
                rows_v.at[p, pl.ds(NT, CH)], gsem)
            pltpu.async_copy(
                t128_hbm.at[idx_t.at[j, pl.ds(CH, CH)]],
                rows_v.at[p, pl.ds(NT + CH, CH)], gsem)

        @pl.when(j >= GDEPTH)
        def _():
            t = j - GDEPTH
            q = lax.rem(t, NBUF)
            drain_gather()
            drain_gather()
            pltpu.async_copy(
                rows_v.at[q, pl.ds(0, S), pl.ds(0, D)],
                out_hbm.at[w * BPW + t], osem)

        return carry

    lax.fori_loop(0, BPW + GDEPTH, body, 0)
    for _ in range(NBUF - GDEPTH):
        drain_out()


@jax.jit
def _gather(tok_t, wte_t, learned_embedding):
    mesh = plsc.VectorSubcoreMesh(core_axis_name="c", subcore_axis_name="s")
    t128 = pl.kernel(
        _tp_body,
        out_type=jax.ShapeDtypeStruct((VPAD, 128), jnp.float32),
        mesh=mesh,
        scratch_types=[
            pltpu.VMEM((128, 128), jnp.float32),
            pltpu.VMEM((256, 128), jnp.float32),
            pltpu.SemaphoreType.DMA,
            pltpu.SemaphoreType.DMA,
        ],
        compiler_params=pltpu.CompilerParams(
            use_tc_tiling_on_sc=True, needs_layout_passes=False,
            disable_bounds_checks=True),
    )(wte_t)
    return pl.kernel(
        _sc_body,
        out_type=jax.ShapeDtypeStruct((B, S, D), jnp.float32),
        mesh=mesh,
        scratch_types=[
            pltpu.VMEM((SG, BPW), jnp.int32),
            pltpu.VMEM((BPW, IW), jnp.int32),
            pltpu.VMEM((NBUF, RV, 128), jnp.float32),
            pltpu.SemaphoreType.DMA,
            pltpu.SemaphoreType.DMA,
        ],
        compiler_params=pltpu.CompilerParams(
            use_tc_tiling_on_sc=False, needs_layout_passes=False),
    )(tok_t, t128, learned_embedding)


def kernel(tokens, wte_weight, learned_embedding):
    tok_t = jnp.swapaxes(tokens, 0, 1)
    wte_t = jnp.swapaxes(wte_weight, 0, 1)
    return _gather(tok_t, wte_t, learned_embedding)


# speedup vs baseline: 1.0963x; 1.0963x over previous
import jax
import jax.numpy as jnp
from jax import lax
from jax.experimental import pallas as pl
from jax.experimental.pallas import tpu as pltpu
from jax.experimental.pallas import tpu_sc as plsc

D = 64
B = 1024
S = 200
NT = 20
SG = S - NT
V = 1000000
NC = 2
NS = 16
NW = NC * NS
BPW = B // NW
CH = 96
IW = 2 * CH
RV = NT + IW
NBUF = 4
GDEPTH = 2

VPAD = 1000064
NBLK = VPAD // 128
BPW0 = (NBLK + NW - 1) // NW


def _tp_body(wte_t_hbm, t128_hbm, tin, tout, isem, osem):
    w = lax.axis_index("s") * NC + lax.axis_index("c")
    base = w * BPW0
    lanes = lax.iota(jnp.int32, 16)

    def drain_o():
        pltpu.make_async_copy(
            t128_hbm.at[pl.ds(0, 128)], tout.at[pl.ds(0, 128)], osem).wait()

    def drain_i():
        pltpu.make_async_copy(
            t128_hbm.at[pl.ds(0, 64)], tin.at[pl.ds(0, 64)], isem).wait()

    for q in range(2):
        @pl.when(base + q < NBLK)
        def _():
            pltpu.async_copy(wte_t_hbm.at[:, pl.ds((base + q) * 128, 128)],
                             tin.at[pl.ds(q * 64, 64)], isem)

    def body(i, carry):
        vh = base + i

        @pl.when(vh < NBLK)
        def _():
            drain_i()

            @pl.when((i < BPW0 - 2) & (vh + 2 < NBLK))
            def _():
                pltpu.async_copy(
                    wte_t_hbm.at[:, pl.ds((vh + 2) * 128, 128)],
                    tin.at[pl.ds(lax.rem(i + 2, 3) * 64, 64)], isem)

            @pl.when(i >= 3)
            def _():
                drain_o()

            pin64 = lax.rem(i, 3) * 64
            pin128 = lax.rem(i, 3) * 128

            @plsc.parallel_loop(0, 16)
            def ebody(j):
                dv0 = (lanes + j) & 15
                for r0 in (0, 16, 32, 48):
                    dv = dv0 + r0
                    dv2 = dv + pin64
                    vals = [plsc.load_gather(tin, [dv2, lanes + cb * 16])
                            for cb in range(8)]
                    for cb in range(8):
                        plsc.store_scatter(
                            tout, [lanes + cb * 16 + pin128, dv], vals[cb])
            pltpu.async_copy(tout.at[pl.ds(pin128, 128)],
                             t128_hbm.at[pl.ds(vh * 128, 128)], osem)

        return carry

    lax.fori_loop(0, BPW0, body, 0)
    for r in (3, 2, 1):
        @pl.when(NBLK - base >= r)
        def _():
            drain_o()


def _sc_body(tok_hbm, t128_hbm, learned_hbm, out_hbm, idx_raw, idx_t, rows_v,
             gsem, osem):
    w = lax.axis_index("s") * NC + lax.axis_index("c")

    def drain_out():
        pltpu.make_async_copy(
            out_hbm.at[0], rows_v.at[0, pl.ds(0, S), pl.ds(0, D)],
            osem).wait()

    def drain_gather():
        pltpu.make_async_copy(
            t128_hbm.at[pl.ds(0, CH)], rows_v.at[0, pl.ds(NT, CH)],
            gsem).wait()

    pltpu.sync_copy(tok_hbm.at[pl.ds(NT, SG), pl.ds(w * BPW, BPW)], idx_raw)
    for k in range(NBUF):
        pltpu.sync_copy(learned_hbm, rows_v.at[k, pl.ds(0, NT), pl.ds(0, D)])

    lanes = lax.iota(jnp.int32, 16)

    def tbody(j, carry):
        colj = jnp.full((16,), j, jnp.int32)
        for k in range(IW // 16):
            rows = jnp.minimum(k * 16 + lanes, SG - 1)
            idx_t[j, pl.ds(k * 16, 16)] = plsc.load_gather(idx_raw, [rows, colj])
        return carry

    lax.fori_loop(0, BPW, tbody, 0)

    def body(j, carry):
        p = lax.rem(j, NBUF)

        @pl.when(j >= NBUF)
        def _():
            drain_out()

        @pl.when(j < BPW)
        def _():
            pltpu.async_copy(
                t128_hbm.at[idx_t.at[j, pl.ds(0, CH)]],
                rows_v.at[p, pl.ds(NT, CH)], gsem)
            pltpu.async_copy(
                t128_hbm.at[idx_t.at[j, pl.ds(CH, CH)]],
                rows_v.at[p, pl.ds(NT + CH, CH)], gsem)

        @pl.when(j >= GDEPTH)
        def _():
            t = j - GDEPTH
            q = lax.rem(t, NBUF)
            drain_gather()
            drain_gather()
            pltpu.async_copy(
                rows_v.at[q, pl.ds(0, S), pl.ds(0, D)],
                out_hbm.at[w * BPW + t], osem)

        return carry

    lax.fori_loop(0, BPW + GDEPTH, body, 0)
    for _ in range(NBUF - GDEPTH):
        drain_out()


@jax.jit
def _gather(tok_t, wte_t, learned_embedding):
    mesh = plsc.VectorSubcoreMesh(core_axis_name="c", subcore_axis_name="s")
    t128 = pl.kernel(
        _tp_body,
        out_type=jax.ShapeDtypeStruct((VPAD, 128), jnp.float32),
        mesh=mesh,
        scratch_types=[
            pltpu.VMEM((192, 128), jnp.float32),
            pltpu.VMEM((384, 128), jnp.float32),
            pltpu.SemaphoreType.DMA,
            pltpu.SemaphoreType.DMA,
        ],
        compiler_params=pltpu.CompilerParams(
            use_tc_tiling_on_sc=True, needs_layout_passes=False,
            disable_bounds_checks=True),
    )(wte_t)
    return pl.kernel(
        _sc_body,
        out_type=jax.ShapeDtypeStruct((B, S, D), jnp.float32),
        mesh=mesh,
        scratch_types=[
            pltpu.VMEM((SG, BPW), jnp.int32),
            pltpu.VMEM((BPW, IW), jnp.int32),
            pltpu.VMEM((NBUF, RV, 128), jnp.float32),
            pltpu.SemaphoreType.DMA,
            pltpu.SemaphoreType.DMA,
        ],
        compiler_params=pltpu.CompilerParams(
            use_tc_tiling_on_sc=False, needs_layout_passes=False),
    )(tok_t, t128, learned_embedding)


def kernel(tokens, wte_weight, learned_embedding):
    tok_t = jnp.swapaxes(tokens, 0, 1)
    wte_t = jnp.swapaxes(wte_weight, 0, 1)
    return _gather(tok_t, wte_t, learned_embedding)


# speedup vs baseline: 1.0979x; 1.0015x over previous
import jax
import jax.numpy as jnp
from jax import lax
from jax.experimental import pallas as pl
from jax.experimental.pallas import tpu as pltpu
from jax.experimental.pallas import tpu_sc as plsc

D = 64
B = 1024
S = 200
NT = 20
SG = S - NT
V = 1000000
NC = 2
NS = 16
NW = NC * NS
BPW = B // NW
CH = 96
IW = 2 * CH
RV = NT + IW
NBUF = 4
GDEPTH = 2

VPAD = 1000064
NBLK = VPAD // 128
BPW0 = (NBLK + NW - 1) // NW


def _tp_body(wte_t_hbm, t128_hbm, tin, tout, isem, osem):
    w = lax.axis_index("s") * NC + lax.axis_index("c")
    base = w * BPW0
    lanes = lax.iota(jnp.int32, 16)

    def drain_o():
        pltpu.make_async_copy(
            t128_hbm.at[pl.ds(0, 128)], tout.at[pl.ds(0, 128)], osem).wait()

    def drain_i():
        pltpu.make_async_copy(
            t128_hbm.at[pl.ds(0, 64)], tin.at[pl.ds(0, 64)], isem).wait()

    for q in range(3):
        @pl.when(base + q < NBLK)
        def _():
            pltpu.async_copy(wte_t_hbm.at[:, pl.ds((base + q) * 128, 128)],
                             tin.at[pl.ds(q * 64, 64)], isem)

    def body(i, carry):
        vh = base + i

        @pl.when(vh < NBLK)
        def _():
            drain_i()

            @pl.when((i < BPW0 - 3) & (vh + 3 < NBLK))
            def _():
                pltpu.async_copy(
                    wte_t_hbm.at[:, pl.ds((vh + 3) * 128, 128)],
                    tin.at[pl.ds(lax.rem(i + 3, 4) * 64, 64)], isem)

            @pl.when(i >= 4)
            def _():
                drain_o()

            pin64 = lax.rem(i, 4) * 64
            pin128 = lax.rem(i, 4) * 128

            @plsc.parallel_loop(0, 16)
            def ebody(j):
                dv0 = (lanes + j) & 15
                for r0 in (0, 16, 32, 48):
                    dv = dv0 + r0
                    dv2 = dv + pin64
                    vals = [plsc.load_gather(tin, [dv2, lanes + cb * 16])
                            for cb in range(8)]
                    for cb in range(8):
                        plsc.store_scatter(
                            tout, [lanes + cb * 16 + pin128, dv], vals[cb])
            pltpu.async_copy(tout.at[pl.ds(pin128, 128)],
                             t128_hbm.at[pl.ds(vh * 128, 128)], osem)

        return carry

    lax.fori_loop(0, BPW0, body, 0)
    for r in (4, 3, 2, 1):
        @pl.when(NBLK - base >= r)
        def _():
            drain_o()


def _sc_body(tok_hbm, t128_hbm, learned_hbm, out_hbm, idx_raw, idx_t, rows_v,
             gsem, osem):
    w = lax.axis_index("s") * NC + lax.axis_index("c")

    def drain_out():
        pltpu.make_async_copy(
            out_hbm.at[0], rows_v.at[0, pl.ds(0, S), pl.ds(0, D)],
            osem).wait()

    def drain_gather():
        pltpu.make_async_copy(
            t128_hbm.at[pl.ds(0, CH)], rows_v.at[0, pl.ds(NT, CH)],
            gsem).wait()

    pltpu.sync_copy(tok_hbm.at[pl.ds(NT, SG), pl.ds(w * BPW, BPW)], idx_raw)
    for k in range(NBUF):
        pltpu.sync_copy(learned_hbm, rows_v.at[k, pl.ds(0, NT), pl.ds(0, D)])

    lanes = lax.iota(jnp.int32, 16)

    def tbody(j, carry):
        colj = jnp.full((16,), j, jnp.int32)
        for k in range(IW // 16):
            rows = jnp.minimum(k * 16 + lanes, SG - 1)
            idx_t[j, pl.ds(k * 16, 16)] = plsc.load_gather(idx_raw, [rows, colj])
        return carry

    lax.fori_loop(0, BPW, tbody, 0)

    def body(j, carry):
        p = lax.rem(j, NBUF)

        @pl.when(j >= NBUF)
        def _():
            drain_out()

        @pl.when(j < BPW)
        def _():
            pltpu.async_copy(
                t128_hbm.at[idx_t.at[j, pl.ds(0, CH)]],
                rows_v.at[p, pl.ds(NT, CH)], gsem)
            pltpu.async_copy(
                t128_hbm.at[idx_t.at[j, pl.ds(CH, CH)]],
                rows_v.at[p, pl.ds(NT + CH, CH)], gsem)

        @pl.when(j >= GDEPTH)
        def _():
            t = j - GDEPTH
            q = lax.rem(t, NBUF)
            drain_gather()
            drain_gather()
            pltpu.async_copy(
                rows_v.at[q, pl.ds(0, S), pl.ds(0, D)],
                out_hbm.at[w * BPW + t], osem)

        return carry

    lax.fori_loop(0, BPW + GDEPTH, body, 0)
    for _ in range(NBUF - GDEPTH):
        drain_out()


@jax.jit
def _gather(tok_t, wte_t, learned_embedding):
    mesh = plsc.VectorSubcoreMesh(core_axis_name="c", subcore_axis_name="s")
    t128 = pl.kernel(
        _tp_body,
        out_type=jax.ShapeDtypeStruct((VPAD, 128), jnp.float32),
        mesh=mesh,
        scratch_types=[
            pltpu.VMEM((256, 128), jnp.float32),
            pltpu.VMEM((512, 128), jnp.float32),
            pltpu.SemaphoreType.DMA,
            pltpu.SemaphoreType.DMA,
        ],
        compiler_params=pltpu.CompilerParams(
            use_tc_tiling_on_sc=True, needs_layout_passes=False,
            disable_bounds_checks=True),
    )(wte_t)
    return pl.kernel(
        _sc_body,
        out_type=jax.ShapeDtypeStruct((B, S, D), jnp.float32),
        mesh=mesh,
        scratch_types=[
            pltpu.VMEM((SG, BPW), jnp.int32),
            pltpu.VMEM((BPW, IW), jnp.int32),
            pltpu.VMEM((NBUF, RV, 128), jnp.float32),
            pltpu.SemaphoreType.DMA,
            pltpu.SemaphoreType.DMA,
        ],
        compiler_params=pltpu.CompilerParams(
            use_tc_tiling_on_sc=False, needs_layout_passes=False),
    )(tok_t, t128, learned_embedding)


def kernel(tokens, wte_weight, learned_embedding):
    tok_t = jnp.swapaxes(tokens, 0, 1)
    wte_t = jnp.swapaxes(wte_weight, 0, 1)
    return _gather(tok_t, wte_t, learned_embedding)
